# baseline (speedup 1.0000x reference)
import functools

import jax
import jax.numpy as jnp
from jax import lax
from jax.experimental import pallas as pl
from jax.experimental.pallas import tpu as pltpu
from jax.experimental.pallas import tpu_sc as plsc

_D = 64
_L = 16
_NC, _NS = 2, 16
_NW = _NC * _NS
_BT = 128
_NSLOT = 4

_SEQ = 200
_BATCH = 4096
_NBT = _BATCH // _BT
_NBLK = _SEQ * _NBT
_BPW = _NBLK // _NW


def _embed_lookup(x4, table):
    mesh = plsc.VectorSubcoreMesh(core_axis_name="c", subcore_axis_name="s")

    @functools.partial(
        pl.kernel,
        out_type=jax.ShapeDtypeStruct((_SEQ, _D // 8, _NBT, 8 * _BT), jnp.float32),
        mesh=mesh,
        scratch_types=[
            pltpu.VMEM((_NSLOT, _BT), jnp.int32),
            pltpu.VMEM((_NSLOT, _BT, _D), jnp.float32),
            pltpu.VMEM((_NSLOT, _D // 8, 8 * _BT), jnp.float32),
            pltpu.SemaphoreType.DMA,
            pltpu.SemaphoreType.DMA,
            pltpu.SemaphoreType.DMA,
            pltpu.SemaphoreType.DMA,
            pltpu.SemaphoreType.DMA,
            pltpu.SemaphoreType.DMA,
            pltpu.SemaphoreType.DMA,
            pltpu.SemaphoreType.DMA,
        ],
        compiler_params=pltpu.CompilerParams(use_tc_tiling_on_sc=False,
                                             needs_layout_passes=False),
    )
    def k(x_hbm, table_hbm, out_hbm, idx_v, rows_v, tbuf, g0, g1, g2, g3,
          o0, o1, o2, o3):
        gsem = (g0, g1, g2, g3)
        osem = (o0, o1, o2, o3)
        wid = lax.axis_index("s") * _NC + lax.axis_index("c")
        blk0 = wid * _BPW
        ii = lax.iota(jnp.int32, _L)

        def stage_and_fire(i, slot):
            s = i // _NBT
            bt = i % _NBT
            pltpu.sync_copy(x_hbm.at[s // 8, bt, s % 8], idx_v.at[slot])
            pltpu.async_copy(table_hbm.at[idx_v.at[slot]], rows_v.at[slot],
                             gsem[slot])

        def transpose_block(slot):
            def per_e(e, carry):
                evec = jnp.full((_L,), 0, jnp.int32) + e
                col = (e % 8) * _BT
                for g in range(_BT // _L):
                    v = plsc.load_gather(rows_v.at[slot], [ii + g * _L, evec])
                    tbuf[slot, e // 8, pl.ds(col + g * _L, _L)] = v * 8.0
                return carry

            lax.fori_loop(0, _D, per_e, 0)

        def fire_out(i, slot):
            s = i // _NBT
            bt = i % _NBT
            for et in range(_D // 8):
                pltpu.async_copy(tbuf.at[slot, et], out_hbm.at[s, et, bt],
                                 osem[slot])

        def wait_out(i, slot):
            s = i // _NBT
            bt = i % _NBT
            for et in range(_D // 8):
                pltpu.make_async_copy(tbuf.at[slot, et], out_hbm.at[s, et, bt],
                                      osem[slot]).wait()

        stage_and_fire(blk0, 0)

        def outer(o, carry):
            for k_ in range(_NSLOT):
                i = o * _NSLOT + k_
                blk = blk0 + i
                slot = k_
                nslot = (k_ + 1) % _NSLOT

                @pl.when(i + 1 < _BPW)
                def _():
                    @pl.when(i + 1 >= _NSLOT)
                    def _():
                        wait_out(blk + 1 - _NSLOT, nslot)

                    stage_and_fire(blk + 1, nslot)

                pltpu.make_async_copy(table_hbm.at[idx_v.at[slot]],
                                      rows_v.at[slot], gsem[slot]).wait()
                transpose_block(slot)
                fire_out(blk, slot)
            return carry

        lax.fori_loop(0, _BPW // _NSLOT, outer, 0)

        for k_ in range(_NSLOT):
            wait_out(blk0 + _BPW - _NSLOT + k_, k_)

    return k(x4, table)


def kernel(x, table):
    x4 = x.T.reshape(_SEQ // 8, 8, _NBT, _BT).transpose(0, 2, 1, 3)
    out4 = _embed_lookup(x4, table)
    out = (out4.reshape(_SEQ, _D // 8, _NBT, 8, _BT)
           .transpose(2, 4, 0, 1, 3)
           .reshape(_BATCH, _SEQ, _D))
    return out


# speedup vs baseline: 1.0399x; 1.0399x over previous
import functools

import jax
import jax.numpy as jnp
from jax import lax
from jax.experimental import pallas as pl
from jax.experimental.pallas import tpu as pltpu
from jax.experimental.pallas import tpu_sc as plsc

_D = 64
_L = 16
_NC, _NS = 2, 16
_NW = _NC * _NS
_BT = 128
_NSLOT = 4

_SEQ = 200
_BATCH = 4096
_NBT = _BATCH // _BT


def _embed_lookup(x4, table):
    mesh = plsc.VectorSubcoreMesh(core_axis_name="c", subcore_axis_name="s")

    @functools.partial(
        pl.kernel,
        out_type=jax.ShapeDtypeStruct((_SEQ, _D // 8, _NBT, 8 * _BT), jnp.float32),
        mesh=mesh,
        scratch_types=[
            pltpu.VMEM((_SEQ // 8, 8, _BT), jnp.int32),
            pltpu.VMEM((_NSLOT, _BT, _D), jnp.float32),
            pltpu.VMEM((_NSLOT, _D // 8, 8 * _BT), jnp.float32),
            pltpu.SemaphoreType.DMA,
            pltpu.SemaphoreType.DMA,
            pltpu.SemaphoreType.DMA,
            pltpu.SemaphoreType.DMA,
            pltpu.SemaphoreType.DMA,
            pltpu.SemaphoreType.DMA,
            pltpu.SemaphoreType.DMA,
            pltpu.SemaphoreType.DMA,
        ],
        compiler_params=pltpu.CompilerParams(use_tc_tiling_on_sc=False,
                                             needs_layout_passes=False),
    )
    def k(x_hbm, table_hbm, out_hbm, idx_all, rows_v, tbuf, g0, g1, g2, g3,
          o0, o1, o2, o3):
        gsem = (g0, g1, g2, g3)
        osem = (o0, o1, o2, o3)
        w = lax.axis_index("s") * _NC + lax.axis_index("c")
        ii = lax.iota(jnp.int32, _L)

        def stage(st, carry):
            pltpu.sync_copy(x_hbm.at[st, w], idx_all.at[st])
            return carry

        lax.fori_loop(0, _SEQ // 8, stage, 0)

        def fire_gather(s, slot):
            pltpu.async_copy(table_hbm.at[idx_all.at[s // 8, s % 8]],
                             rows_v.at[slot], gsem[slot])

        def transpose_block(slot):
            def per_et(et, carry):
                ebase = jnp.full((_L,), 0, jnp.int32) + et * 8
                for em in range(8):
                    evec = ebase + em
                    for g in range(_BT // _L):
                        v = plsc.load_gather(rows_v.at[slot],
                                             [ii + g * _L, evec])
                        tbuf[slot, et, pl.ds(em * _BT + g * _L, _L)] = v * 8.0
                return carry

            lax.fori_loop(0, _D // 8, per_et, 0)

        def fire_out(s, slot):
            pltpu.async_copy(tbuf.at[slot], out_hbm.at[s, :, w], osem[slot])

        def wait_out(s, slot):
            pltpu.make_async_copy(tbuf.at[slot], out_hbm.at[s, :, w],
                                  osem[slot]).wait()

        fire_gather(0, 0)
        fire_gather(1, 1)

        def outer(o, carry):
            for k_ in range(_NSLOT):
                s = o * _NSLOT + k_
                slot = k_
                pslot = (k_ + 2) % _NSLOT

                @pl.when(s + 2 < _SEQ)
                def _():
                    @pl.when(s + 2 >= _NSLOT)
                    def _():
                        wait_out(s + 2 - _NSLOT, pslot)

                    fire_gather(s + 2, pslot)

                pltpu.make_async_copy(
                    table_hbm.at[idx_all.at[s // 8, s % 8]],
                    rows_v.at[slot], gsem[slot]).wait()
                transpose_block(slot)
                fire_out(s, slot)
            return carry

        lax.fori_loop(0, _SEQ // _NSLOT, outer, 0)

        for k_ in range(_NSLOT):
            wait_out(_SEQ - _NSLOT + k_, k_)

    return k(x4, table)


def kernel(x, table):
    x4 = x.T.reshape(_SEQ // 8, 8, _NBT, _BT).transpose(0, 2, 1, 3)
    out4 = _embed_lookup(x4, table)
    out = (out4.reshape(_SEQ, _D // 8, _NBT, 8, _BT)
           .transpose(2, 4, 0, 1, 3)
           .reshape(_BATCH, _SEQ, _D))
    return out


# speedup vs baseline: 1.3796x; 1.3267x over previous
import functools

import jax
import jax.numpy as jnp
from jax import lax
from jax.experimental import pallas as pl
from jax.experimental.pallas import tpu as pltpu
from jax.experimental.pallas import tpu_sc as plsc

_D = 64
_L = 16
_NC, _NS = 2, 16
_NW = _NC * _NS
_BT = 128
_NSLOT = 4

_SEQ = 200
_BATCH = 4096
_NBT = _BATCH // _BT


def _embed_lookup(x4, table):
    mesh = plsc.VectorSubcoreMesh(core_axis_name="c", subcore_axis_name="s")

    @functools.partial(
        pl.kernel,
        out_type=jax.ShapeDtypeStruct((_SEQ, _D // 8, _NBT, 8 * _BT), jnp.float32),
        mesh=mesh,
        scratch_types=[
            pltpu.VMEM((_SEQ // 8, 8, _BT), jnp.int32),
            pltpu.VMEM((_NSLOT, _BT, _D), jnp.float32),
            pltpu.VMEM((_NSLOT, _D // 8, 8 * _BT), jnp.float32),
            pltpu.SemaphoreType.DMA,
            pltpu.SemaphoreType.DMA,
            pltpu.SemaphoreType.DMA,
            pltpu.SemaphoreType.DMA,
            pltpu.SemaphoreType.DMA,
            pltpu.SemaphoreType.DMA,
            pltpu.SemaphoreType.DMA,
            pltpu.SemaphoreType.DMA,
        ],
        compiler_params=pltpu.CompilerParams(use_tc_tiling_on_sc=False,
                                             needs_layout_passes=False),
    )
    def k(x_hbm, table_hbm, out_hbm, idx_all, rows_v, tbuf, g0, g1, g2, g3,
          o0, o1, o2, o3):
        gsem = (g0, g1, g2, g3)
        osem = (o0, o1, o2, o3)
        w = lax.axis_index("s") * _NC + lax.axis_index("c")
        ii = lax.iota(jnp.int32, _L)

        def stage(st, carry):
            pltpu.sync_copy(x_hbm.at[st, w], idx_all.at[st])
            return carry

        lax.fori_loop(0, _SEQ // 8, stage, 0)

        def fire_gather(s, slot):
            pltpu.async_copy(table_hbm.at[idx_all.at[s // 8, s % 8]],
                             rows_v.at[slot], gsem[slot])

        def transpose_block(slot):
            @plsc.parallel_loop(0, _D, 1, unroll=2)
            def per_e(e):
                evec = jnp.full((_L,), 0, jnp.int32) + e
                col = (e % 8) * _BT
                vs = [plsc.load_gather(rows_v.at[slot], [ii + g * _L, evec])
                      for g in range(_BT // _L)]
                for g in range(_BT // _L):
                    tbuf[slot, e // 8, pl.ds(col + g * _L, _L)] = vs[g] * 8.0

        def fire_out(s, slot):
            pltpu.async_copy(tbuf.at[slot], out_hbm.at[s, :, w], osem[slot])

        def wait_out(s, slot):
            pltpu.make_async_copy(tbuf.at[slot], out_hbm.at[s, :, w],
                                  osem[slot]).wait()

        fire_gather(0, 0)
        fire_gather(1, 1)

        def outer(o, carry):
            for k_ in range(_NSLOT):
                s = o * _NSLOT + k_
                slot = k_
                pslot = (k_ + 2) % _NSLOT

                @pl.when(s + 2 < _SEQ)
                def _():
                    @pl.when(s + 2 >= _NSLOT)
                    def _():
                        wait_out(s + 2 - _NSLOT, pslot)

                    fire_gather(s + 2, pslot)

                pltpu.make_async_copy(
                    table_hbm.at[idx_all.at[s // 8, s % 8]],
                    rows_v.at[slot], gsem[slot]).wait()
                transpose_block(slot)
                fire_out(s, slot)
            return carry

        lax.fori_loop(0, _SEQ // _NSLOT, outer, 0)

        for k_ in range(_NSLOT):
            wait_out(_SEQ - _NSLOT + k_, k_)

    return k(x4, table)


def kernel(x, table):
    x4 = x.T.reshape(_SEQ // 8, 8, _NBT, _BT).transpose(0, 2, 1, 3)
    out4 = _embed_lookup(x4, table)
    out = (out4.reshape(_SEQ, _D // 8, _NBT, 8, _BT)
           .transpose(2, 4, 0, 1, 3)
           .reshape(_BATCH, _SEQ, _D))
    return out
